# baseline (speedup 1.0000x reference)
import functools

import jax
import jax.numpy as jnp
from jax.experimental import pallas as pl

M_LIST = (72, 72, 72, 72, 43, 43)
N_SORT = 512
N_SEL = 518
N_OUT = 519
N_REM = 58
INF = float("inf")
BIG = 1e6


def _bitonic_sort(key, ids, lane):
    pos = lane.astype(jnp.float32)
    size = 2
    while size <= N_SORT:
        stride = size // 2
        while stride >= 1:
            lo = (lane & stride) == 0
            k2 = jnp.where(lo, jnp.roll(key, -stride, axis=1),
                           jnp.roll(key, stride, axis=1))
            p2 = jnp.where(lo, jnp.roll(pos, -stride, axis=1),
                           jnp.roll(pos, stride, axis=1))
            i2 = jnp.where(lo, jnp.roll(ids, -stride, axis=1),
                           jnp.roll(ids, stride, axis=1))
            up = (lane & size) == 0
            me_lt = (key < k2) | ((key == k2) & (pos < p2))
            keep = me_lt == (lo == up)
            key = jnp.where(keep, key, k2)
            pos = jnp.where(keep, pos, p2)
            ids = jnp.where(keep, ids, i2)
            stride //= 2
        size *= 2
    return key, ids


def _sort_body(u, sc_ref, out_ref):
    sc = sc_ref[0]
    nm = jnp.max(sc, axis=1, keepdims=True)
    lane = jax.lax.broadcasted_iota(jnp.int32, (1, N_SORT), 1)
    subu = jax.lax.broadcasted_iota(jnp.int32, (sc.shape[0], 1), 0)
    lane_u = jax.lax.broadcasted_iota(jnp.int32, (1, sc.shape[0]), 1)
    nm_lane = jnp.sum((subu == lane_u).astype(jnp.float32) * nm,
                      axis=0, keepdims=True)
    key = jnp.concatenate(
        [nm_lane, jnp.full((1, N_SORT - u), INF, jnp.float32)], axis=1)
    ids = lane.astype(jnp.float32)
    _, ids = _bitonic_sort(key, ids, lane)
    out_ref[0] = ids[:, :u].astype(jnp.int32)


def _final_body(x_ref, scT_ref, selid_ref, unselid_ref, xout_ref, idx_ref):
    xt = x_ref[0]
    sc = scT_ref[0]
    lane518 = jax.lax.broadcasted_iota(jnp.int32, (1, N_SEL), 1)
    mv = jnp.max(sc, axis=1, keepdims=True)
    tgt_sub = jnp.min(jnp.where(sc == mv, lane518.astype(jnp.float32), BIG),
                      axis=1, keepdims=True)
    sub58_ = jax.lax.broadcasted_iota(jnp.int32, (N_REM, 1), 0)
    lane58_ = jax.lax.broadcasted_iota(jnp.int32, (1, N_REM), 1)
    tgt58 = jnp.sum((sub58_ == lane58_).astype(jnp.float32) * tgt_sub,
                    axis=0, keepdims=True)

    r519 = jax.lax.broadcasted_iota(jnp.int32, (N_OUT, 1), 0)
    r519f = r519.astype(jnp.float32)
    col = jax.lax.broadcasted_iota(jnp.int32, (1, 577), 1).astype(jnp.float32)
    lane518 = jax.lax.broadcasted_iota(jnp.int32, (1, N_SEL), 1)
    lane58 = jax.lax.broadcasted_iota(jnp.int32, (1, N_REM), 1)

    selid = selid_ref[0].astype(jnp.float32)
    I_sh = (r519 == lane518 + 1).astype(jnp.float32)
    selcol = jnp.sum(I_sh * (selid + 1.0), axis=1, keepdims=True)
    W1 = (col == selcol).astype(jnp.float32)

    A = (r519f == 1.0 + tgt58).astype(jnp.float32)
    unselid = unselid_ref[0].astype(jnp.float32)
    sub58 = jax.lax.broadcasted_iota(jnp.int32, (N_REM, 1), 0)
    I58 = (sub58 == lane58).astype(jnp.float32)
    unsel_sub = jnp.sum(I58 * unselid, axis=1, keepdims=True)
    Bm = (col == unsel_sub + 1.0).astype(jnp.float32)
    Wm = jax.lax.dot_general(A, Bm, (((1,), (0,)), ((), ())))
    count = 1.0 + jnp.sum(A, axis=1, keepdims=True)
    xout = jax.lax.dot_general(
        W1 + Wm, xt, (((1,), (0,)), ((), ())),
        precision=jax.lax.Precision.HIGHEST) / count
    xout_ref[0] = xout
    idx_ref[0] = jnp.broadcast_to(selcol.astype(jnp.int32), (N_OUT, 768))


def _argsort_call(sc):
    sc = jax.lax.optimization_barrier(sc)
    B, U, S = sc.shape
    edge = pl.pallas_call(
        functools.partial(_sort_body, U),
        grid=(B,),
        in_specs=[pl.BlockSpec((1, U, S), lambda b: (b, 0, 0))],
        out_specs=pl.BlockSpec((1, 1, U), lambda b: (b, 0, 0)),
        out_shape=jax.ShapeDtypeStruct((B, 1, U), jnp.int32),
    )(sc)[:, 0]
    return jax.lax.optimization_barrier(edge)


def kernel(x):
    B, T, D = x.shape
    N = T - 1
    toks = x[:, 1:]
    sel_list = [i for i in range(N) if i % 4 == 0]
    unsel_list = [i for i in range(N) if i % 4 != 0]
    sel_idx = jnp.broadcast_to(
        jnp.array(sel_list, jnp.int32)[None], (B, len(sel_list)))
    unsel_idx = jnp.broadcast_to(
        jnp.array(unsel_list, jnp.int32)[None], (B, len(unsel_list)))
    sel_tok = jnp.take_along_axis(toks, sel_idx[:, :, None], axis=1)
    unsel_tok = jnp.take_along_axis(toks, unsel_idx[:, :, None], axis=1)
    for m in M_LIST:
        sel_n = sel_tok / jnp.linalg.norm(sel_tok, axis=-1, keepdims=True)
        unsel_n = unsel_tok / jnp.linalg.norm(unsel_tok, axis=-1, keepdims=True)
        scores = jnp.einsum('bud,bsd->bus', unsel_n, sel_n)
        edge = _argsort_call(scores)
        add_i = edge[:, :m]
        unadd_i = edge[:, m:]
        add_idx = jnp.take_along_axis(unsel_idx, add_i, axis=1)
        add_tok = jnp.take_along_axis(unsel_tok, add_i[:, :, None], axis=1)
        sel_idx = jnp.concatenate([sel_idx, add_idx], axis=1)
        sel_tok = jnp.concatenate([sel_tok, add_tok], axis=1)
        unsel_idx = jnp.take_along_axis(unsel_idx, unadd_i, axis=1)
        unsel_tok = jnp.take_along_axis(unsel_tok, unadd_i[:, :, None], axis=1)
    sel_n = sel_tok / jnp.linalg.norm(sel_tok, axis=-1, keepdims=True)
    unsel_n = unsel_tok / jnp.linalg.norm(unsel_tok, axis=-1, keepdims=True)
    scores = jnp.einsum('bud,bsd->bus', unsel_n, sel_n)
    xb, scores, sel_b, unsel_b = jax.lax.optimization_barrier(
        (x, scores, sel_idx[:, None, :], unsel_idx[:, None, :]))
    return pl.pallas_call(
        _final_body,
        grid=(B,),
        in_specs=[pl.BlockSpec((1, T, D), lambda b: (b, 0, 0)),
                  pl.BlockSpec((1, N_REM, N_SEL), lambda b: (b, 0, 0)),
                  pl.BlockSpec((1, 1, N_SEL), lambda b: (b, 0, 0)),
                  pl.BlockSpec((1, 1, N_REM), lambda b: (b, 0, 0))],
        out_specs=[pl.BlockSpec((1, N_OUT, D), lambda b: (b, 0, 0)),
                   pl.BlockSpec((1, N_OUT, D), lambda b: (b, 0, 0))],
        out_shape=[jax.ShapeDtypeStruct((B, N_OUT, D), jnp.float32),
                   jax.ShapeDtypeStruct((B, N_OUT, D), jnp.int32)],
    )(xb, scores, sel_b, unsel_b)


# speedup vs baseline: 1.7048x; 1.7048x over previous
import functools

import jax
import jax.numpy as jnp
from jax.experimental import pallas as pl

M_LIST = (72, 72, 72, 72, 43, 43)
N_SORT = 512
N_SEL = 518
N_OUT = 519
N_REM = 58
INF = float("inf")
BIG = 1e6


def _bitonic_argsort(key, lane):
    pos = jnp.broadcast_to(lane.astype(jnp.float32), key.shape)
    size = 2
    while size <= N_SORT:
        stride = size // 2
        while stride >= 1:
            lo = (lane & stride) == 0
            k2 = jnp.where(lo, jnp.roll(key, -stride, axis=1),
                           jnp.roll(key, stride, axis=1))
            p2 = jnp.where(lo, jnp.roll(pos, -stride, axis=1),
                           jnp.roll(pos, stride, axis=1))
            up = (lane & size) == 0
            me_lt = (key < k2) | ((key == k2) & (pos < p2))
            keep = me_lt == (lo == up)
            key = jnp.where(keep, key, k2)
            pos = jnp.where(keep, pos, p2)
            stride //= 2
        size *= 2
    return pos


def _sort_body(u, sc_ref, out_ref):
    sc = sc_ref[...]
    nm = jnp.max(sc, axis=2)
    b = nm.shape[0]
    lane = jax.lax.broadcasted_iota(jnp.int32, (1, N_SORT), 1)
    key = jnp.concatenate(
        [nm, jnp.full((b, N_SORT - u), INF, jnp.float32)], axis=1)
    pos = _bitonic_argsort(key, lane)
    out_ref[...] = pos[:, :u].astype(jnp.int32)


def _final_body(x_ref, scT_ref, selid_ref, unselid_ref, xout_ref, idx_ref):
    xt = x_ref[0]
    sc = scT_ref[0]
    lane518 = jax.lax.broadcasted_iota(jnp.int32, (1, N_SEL), 1)
    mv = jnp.max(sc, axis=1, keepdims=True)
    tgt_sub = jnp.min(jnp.where(sc == mv, lane518.astype(jnp.float32), BIG),
                      axis=1, keepdims=True)
    sub58_ = jax.lax.broadcasted_iota(jnp.int32, (N_REM, 1), 0)
    lane58_ = jax.lax.broadcasted_iota(jnp.int32, (1, N_REM), 1)
    tgt58 = jnp.sum((sub58_ == lane58_).astype(jnp.float32) * tgt_sub,
                    axis=0, keepdims=True)

    r519 = jax.lax.broadcasted_iota(jnp.int32, (N_OUT, 1), 0)
    r519f = r519.astype(jnp.float32)
    col = jax.lax.broadcasted_iota(jnp.int32, (1, 577), 1).astype(jnp.float32)
    lane518 = jax.lax.broadcasted_iota(jnp.int32, (1, N_SEL), 1)
    lane58 = jax.lax.broadcasted_iota(jnp.int32, (1, N_REM), 1)

    selid = selid_ref[0].astype(jnp.float32)
    I_sh = (r519 == lane518 + 1).astype(jnp.float32)
    selcol = jnp.sum(I_sh * (selid + 1.0), axis=1, keepdims=True)
    W1 = (col == selcol).astype(jnp.float32)

    A = (r519f == 1.0 + tgt58).astype(jnp.float32)
    unselid = unselid_ref[0].astype(jnp.float32)
    sub58 = jax.lax.broadcasted_iota(jnp.int32, (N_REM, 1), 0)
    I58 = (sub58 == lane58).astype(jnp.float32)
    unsel_sub = jnp.sum(I58 * unselid, axis=1, keepdims=True)
    Bm = (col == unsel_sub + 1.0).astype(jnp.float32)
    Wm = jax.lax.dot_general(A, Bm, (((1,), (0,)), ((), ())))
    count = 1.0 + jnp.sum(A, axis=1, keepdims=True)
    xout = jax.lax.dot_general(
        W1 + Wm, xt, (((1,), (0,)), ((), ()))) / count
    xout_ref[0] = xout
    idx_ref[0] = jnp.broadcast_to(selcol.astype(jnp.int32), (N_OUT, 768))


def _argsort_call(sc):
    sc = jax.lax.optimization_barrier(sc)
    B, U, S = sc.shape
    edge = pl.pallas_call(
        functools.partial(_sort_body, U),
        out_shape=jax.ShapeDtypeStruct((B, U), jnp.int32),
    )(sc)
    return jax.lax.optimization_barrier(edge)


def kernel(x):
    B, T, D = x.shape
    N = T - 1
    toks = x[:, 1:]
    sel_list = [i for i in range(N) if i % 4 == 0]
    unsel_list = [i for i in range(N) if i % 4 != 0]
    sel_idx = jnp.broadcast_to(
        jnp.array(sel_list, jnp.int32)[None], (B, len(sel_list)))
    unsel_idx = jnp.broadcast_to(
        jnp.array(unsel_list, jnp.int32)[None], (B, len(unsel_list)))
    sel_tok = jnp.take_along_axis(toks, sel_idx[:, :, None], axis=1)
    unsel_tok = jnp.take_along_axis(toks, unsel_idx[:, :, None], axis=1)
    for m in M_LIST:
        sel_n = sel_tok / jnp.linalg.norm(sel_tok, axis=-1, keepdims=True)
        unsel_n = unsel_tok / jnp.linalg.norm(unsel_tok, axis=-1, keepdims=True)
        scores = jnp.einsum('bud,bsd->bus', unsel_n, sel_n)
        edge = _argsort_call(scores)
        add_i = edge[:, :m]
        unadd_i = edge[:, m:]
        add_idx = jnp.take_along_axis(unsel_idx, add_i, axis=1)
        add_tok = jnp.take_along_axis(unsel_tok, add_i[:, :, None], axis=1)
        sel_idx = jnp.concatenate([sel_idx, add_idx], axis=1)
        sel_tok = jnp.concatenate([sel_tok, add_tok], axis=1)
        unsel_idx = jnp.take_along_axis(unsel_idx, unadd_i, axis=1)
        unsel_tok = jnp.take_along_axis(unsel_tok, unadd_i[:, :, None], axis=1)
    sel_n = sel_tok / jnp.linalg.norm(sel_tok, axis=-1, keepdims=True)
    unsel_n = unsel_tok / jnp.linalg.norm(unsel_tok, axis=-1, keepdims=True)
    scores = jnp.einsum('bud,bsd->bus', unsel_n, sel_n)
    xb, scores, sel_b, unsel_b = jax.lax.optimization_barrier(
        (x, scores, sel_idx[:, None, :], unsel_idx[:, None, :]))
    return pl.pallas_call(
        _final_body,
        grid=(B,),
        in_specs=[pl.BlockSpec((1, T, D), lambda b: (b, 0, 0)),
                  pl.BlockSpec((1, N_REM, N_SEL), lambda b: (b, 0, 0)),
                  pl.BlockSpec((1, 1, N_SEL), lambda b: (b, 0, 0)),
                  pl.BlockSpec((1, 1, N_REM), lambda b: (b, 0, 0))],
        out_specs=[pl.BlockSpec((1, N_OUT, D), lambda b: (b, 0, 0)),
                   pl.BlockSpec((1, N_OUT, D), lambda b: (b, 0, 0))],
        out_shape=[jax.ShapeDtypeStruct((B, N_OUT, D), jnp.float32),
                   jax.ShapeDtypeStruct((B, N_OUT, D), jnp.int32)],
    )(xb, scores, sel_b, unsel_b)
